# baseline (speedup 1.0000x reference)
import jax
import jax.numpy as jnp
from jax import lax
from jax.experimental import pallas as pl
from jax.experimental.pallas import tpu as pltpu
from jax.experimental.pallas import tpu_sc as plsc

f32 = jnp.float32
i32 = jnp.int32

N = 10000
E = 320000
H = 128
NC = 2
NS = 16
NW = NC * NS
NR = 320
NPAD = NW * NR
CAP = 12320
SCH = 4000
NSCH = E // SCH
ECH = 80
CAPCH = CAP // ECH

_SC_PARAMS = pltpu.CompilerParams(needs_layout_passes=False)


def _silu(v):
    return v * jax.nn.sigmoid(v)


def _sc_mesh():
    return plsc.VectorSubcoreMesh(
        core_axis_name="c", subcore_axis_name="s",
        num_cores=NC, num_subcores=NS)


def _sc_prep_body(row_h, col_h, px_h, py_h, pz_h,
                  krow_h, kcol_h, kdist_h, kcnt_h, cnt_h,
                  pxv, pyv, pzv, rsc, csc, krow_st, kcol_st, kdist_st,
                  cntv, sbuf, sem_sc):
    w = lax.axis_index("c") * NS + lax.axis_index("s")
    lo = w * NR
    hi = lo + NR
    pltpu.sync_copy(px_h, pxv)
    pltpu.sync_copy(py_h, pyv)
    pltpu.sync_copy(pz_h, pzv)

    z16f = jnp.zeros((16,), f32)
    z16i = jnp.zeros((16,), i32)
    ones16 = jnp.ones((16,), f32)

    def zero_st(i, _):
        sl = pl.ds(i * 16, 16)
        krow_st[sl] = z16i
        kcol_st[sl] = z16i
        kdist_st[sl] = z16f
        return 0
    lax.fori_loop(0, CAP // 16, zero_st, 0)

    def zero_cnt(i, _):
        cntv[pl.ds(i * 16, 16)] = z16f
        return 0
    lax.fori_loop(0, NR // 16, zero_cnt, 0)

    def issue_scan(j, sl):
        off = j * SCH
        pltpu.async_copy(row_h.at[pl.ds(off, SCH)], rsc[sl], sem_sc[sl])
        pltpu.async_copy(col_h.at[pl.ds(off, SCH)], csc[sl], sem_sc[sl])

    def wait_scan(sl):
        pltpu.make_async_copy(row_h.at[pl.ds(0, SCH)], rsc[sl],
                              sem_sc[sl]).wait()
        pltpu.make_async_copy(col_h.at[pl.ds(0, SCH)], csc[sl],
                              sem_sc[sl]).wait()

    issue_scan(0, 0)

    def pair(p, cur):
        for u in range(2):
            j = p * 2 + u

            @pl.when(j + 1 < NSCH)
            def _():
                issue_scan(j + 1, 1 - u)
            wait_scan(u)

            def vec(v, cur):
                sl = pl.ds(v * 16, 16)
                r16 = rsc[u][sl]
                c16 = csc[u][sl]
                mask = jnp.logical_and(c16 >= lo, c16 < hi)
                plsc.addupdate_scatter(cntv, [c16 - lo], ones16, mask=mask)
                mp = plsc.all_reduce_population_count(mask)
                rank = plsc.cumsum(mask.astype(i32)) - 1
                addr = rank + cur
                plsc.store_scatter(krow_st, [addr], r16, mask=mask)
                plsc.store_scatter(kcol_st, [addr], c16, mask=mask)
                return jnp.minimum(cur + mp[0], CAP - 16)
            cur = lax.fori_loop(0, SCH // 16, vec, cur)
        return cur
    kept = lax.fori_loop(0, NSCH // 2, pair, jnp.int32(0))

    def dvec(v, _):
        sl = pl.ds(v * 16, 16)
        r16 = krow_st[sl]
        c16 = kcol_st[sl]
        dx = plsc.load_gather(pxv, [r16]) - plsc.load_gather(pxv, [c16])
        dy = plsc.load_gather(pyv, [r16]) - plsc.load_gather(pyv, [c16])
        dz = plsc.load_gather(pzv, [r16]) - plsc.load_gather(pzv, [c16])
        s = dx * dx + dy * dy + dz * dz + 1e-12
        y = plsc.bitcast(0x5F3759DF - (plsc.bitcast(s, i32) >> 1), f32)
        y = y * (1.5 - 0.5 * s * y * y)
        y = y * (1.5 - 0.5 * s * y * y)
        y = y * (1.5 - 0.5 * s * y * y)
        kdist_st[sl] = s * y
        return 0
    lax.fori_loop(0, CAP // 16, dvec, 0)

    pltpu.sync_copy(krow_st, krow_h.at[pl.ds(w * CAP, CAP)])
    pltpu.sync_copy(kcol_st, kcol_h.at[pl.ds(w * CAP, CAP)])
    pltpu.sync_copy(kdist_st, kdist_h.at[pl.ds(w * CAP, CAP)])
    pltpu.sync_copy(cntv, cnt_h.at[pl.ds(w * NR, NR)])
    sbuf[...] = jnp.full((16,), kept, dtype=i32)
    pltpu.sync_copy(sbuf, kcnt_h.at[pl.ds(w * 16, 16)])


def _sc_prep(row, col, px, py, pz):
    fn = pl.kernel(
        _sc_prep_body,
        out_type=(jax.ShapeDtypeStruct((NW * CAP,), i32),
                  jax.ShapeDtypeStruct((NW * CAP,), i32),
                  jax.ShapeDtypeStruct((NW * CAP,), f32),
                  jax.ShapeDtypeStruct((NW * 16,), i32),
                  jax.ShapeDtypeStruct((NPAD,), f32)),
        mesh=_sc_mesh(),
        compiler_params=_SC_PARAMS,
        scratch_types=[
            pltpu.VMEM((N,), f32), pltpu.VMEM((N,), f32), pltpu.VMEM((N,), f32),
            [pltpu.VMEM((SCH,), i32) for _ in range(2)],
            [pltpu.VMEM((SCH,), i32) for _ in range(2)],
            pltpu.VMEM((CAP,), i32), pltpu.VMEM((CAP,), i32),
            pltpu.VMEM((CAP,), f32),
            pltpu.VMEM((NR,), f32), pltpu.VMEM((16,), i32),
            [pltpu.SemaphoreType.DMA for _ in range(2)],
        ],
    )
    return fn(row, col, px, py, pz)


def _sc_edge_body(a_h, b_h, krow_h, kcol_h, kdist_h, kcnt_h, w1c_h, accp_h,
                  aloc, accl, bbuf, rowv, colv, distv, w1cv, cntb,
                  sem_i, sem_g):
    w = lax.axis_index("c") * NS + lax.axis_index("s")
    pltpu.sync_copy(w1c_h, w1cv)
    pltpu.sync_copy(a_h.at[pl.ds(w * NR * H, NR * H)], aloc)
    pltpu.sync_copy(kcnt_h.at[pl.ds(w * 16, 16)], cntb)
    kcnt = cntb[pl.ds(0, 16)][0]
    nch = (kcnt + (ECH - 1)) // ECH

    z16f = jnp.zeros((16,), f32)

    def zero_acc(i, _):
        accl[pl.ds(i * 16, 16)] = z16f
        return 0
    lax.fori_loop(0, (NR * H) // 16, zero_acc, 0)

    base = w * CAP

    def issue_idx(ch, sl):
        off = base + ch * ECH
        pltpu.async_copy(krow_h.at[pl.ds(off, ECH)], rowv[sl], sem_i[sl])
        pltpu.async_copy(kcol_h.at[pl.ds(off, ECH)], colv[sl], sem_i[sl])
        pltpu.async_copy(kdist_h.at[pl.ds(off, ECH)], distv[sl], sem_i[sl])

    def wait_idx(sl):
        pltpu.make_async_copy(krow_h.at[pl.ds(0, ECH)], rowv[sl],
                              sem_i[sl]).wait()
        pltpu.make_async_copy(kcol_h.at[pl.ds(0, ECH)], colv[sl],
                              sem_i[sl]).wait()
        pltpu.make_async_copy(kdist_h.at[pl.ds(0, ECH)], distv[sl],
                              sem_i[sl]).wait()

    def issue_gather(dsl, isl):
        pltpu.async_copy(b_h.at[rowv[isl]], bbuf[dsl], sem_g[dsl])

    def wait_gather(dsl, isl):
        pltpu.make_async_copy(b_h.at[rowv[isl]], bbuf[dsl],
                              sem_g[dsl]).wait()

    iota16 = lax.iota(i32, 16)
    lobase = w * NR

    def compute(c, dsl, isl):
        kb = jnp.minimum(kcnt - c * ECH, ECH)

        def erow(e, _):
            e16 = jnp.full((16,), e, dtype=i32)
            lcv = plsc.load_gather(colv[isl], [e16]) - lobase
            d16 = plsc.load_gather(distv[isl], [e16])
            abase = lcv * H + iota16
            for k in range(H // 16):
                slc = pl.ds(16 * k, 16)
                addr = abase + (16 * k)
                av = plsc.load_gather(aloc, [addr])
                v = av + bbuf[dsl][e, slc] + d16 * w1cv[slc]
                sg = 1.0 / (1.0 + jnp.exp(-v))
                plsc.addupdate_scatter(accl, [addr], v * sg)
            return 0
        lax.fori_loop(0, kb, erow, 0)

    issue_idx(0, 0)
    issue_idx(1, 1)
    wait_idx(0)
    issue_gather(0, 0)

    def block(i, _):
        for u in range(6):
            c = i * 6 + u
            d0 = u % 2
            d1 = (u + 1) % 2
            i0 = u % 3
            i1 = (u + 1) % 3
            i2 = (u + 2) % 3

            @pl.when(c + 2 < nch)
            def _():
                issue_idx(c + 2, i2)

            @pl.when(c + 1 < nch)
            def _():
                wait_idx(i1)
                issue_gather(d1, i1)

            @pl.when(c < nch)
            def _():
                wait_gather(d0, i0)
                compute(c, d0, i0)
        return 0
    lax.fori_loop(0, (CAPCH + 6) // 6, block, 0)

    pltpu.sync_copy(accl, accp_h.at[pl.ds(w * NR * H, NR * H)])


def _sc_edge(a_flat, b_tab, krow, kcol, kdist, kcnt, w1c):
    fn = pl.kernel(
        _sc_edge_body,
        out_type=jax.ShapeDtypeStruct((NPAD * H,), f32),
        mesh=_sc_mesh(),
        compiler_params=_SC_PARAMS,
        scratch_types=[
            pltpu.VMEM((NR * H,), f32),
            pltpu.VMEM((NR * H,), f32),
            [pltpu.VMEM((ECH, H), f32) for _ in range(2)],
            [pltpu.VMEM((ECH,), i32) for _ in range(3)],
            [pltpu.VMEM((ECH,), i32) for _ in range(3)],
            [pltpu.VMEM((ECH,), f32) for _ in range(3)],
            pltpu.VMEM((H,), f32),
            pltpu.VMEM((16,), i32),
            [pltpu.SemaphoreType.DMA for _ in range(3)],
            [pltpu.SemaphoreType.DMA for _ in range(2)],
        ],
    )
    return fn(a_flat, b_tab, krow, kcol, kdist, kcnt, w1c)


def _tc_pre_body(x_r, temb_r, tw1t_r, tb1_r, tw2t_r, tb2_r,
                 w1at_r, b1_r, w1bt_r, h_o, a_o, b_o):
    t = _silu(jnp.dot(temb_r[...], tw1t_r[...], preferred_element_type=f32)
              + tb1_r[...])
    tp = jnp.dot(t, tw2t_r[...], preferred_element_type=f32) + tb2_r[...]
    h = x_r[...] + tp
    h_o[...] = h
    a = jnp.dot(h, w1at_r[...], preferred_element_type=f32) + b1_r[...]
    a_o[...] = jnp.concatenate([a, jnp.zeros((NPAD - N, H), f32)], axis=0)
    b_o[...] = jnp.dot(h, w1bt_r[...], preferred_element_type=f32)


def _tc_pre(x, time_emb, tw1t, tb1, tw2t, tb2, w1at, b1, w1bt):
    return pl.pallas_call(
        _tc_pre_body,
        out_shape=(jax.ShapeDtypeStruct((N, H), f32),
                   jax.ShapeDtypeStruct((NPAD, H), f32),
                   jax.ShapeDtypeStruct((N, H), f32)),
    )(x, time_emb, tw1t, tb1, tw2t, tb2, w1at, b1, w1bt)


def _layer_core(h_r, accp_r, cnt_r, ew2t_r, eb2_r, nw1t_r, nb1_r,
                nw2t_r, nb2_r, g_r, b_r):
    h = h_r[...]
    inv = 1.0 / jnp.maximum(cnt_r[...][:N], 1.0)
    acc = accp_r[...][:N] * inv
    agg = jnp.dot(acc, ew2t_r[...], preferred_element_type=f32) + eb2_r[...]
    t = _silu(jnp.dot(h, nw1t_r[...], preferred_element_type=f32) + nb1_r[...])
    nm = jnp.dot(t, nw2t_r[...], preferred_element_type=f32) + nb2_r[...]
    y = h + nm + agg
    mu = jnp.mean(y, axis=-1, keepdims=True)
    yc = y - mu
    var = jnp.mean(yc * yc, axis=-1, keepdims=True)
    return yc * lax.rsqrt(var + 1e-5) * g_r[...] + b_r[...]


def _tc_layer_ab_body(h_r, accp_r, cnt_r, ew2t_r, eb2_r, nw1t_r, nb1_r,
                      nw2t_r, nb2_r, g_r, b_r, w1at_r, b1_r, w1bt_r,
                      h_o, a_o, b_o):
    hn = _layer_core(h_r, accp_r, cnt_r, ew2t_r, eb2_r, nw1t_r, nb1_r,
                     nw2t_r, nb2_r, g_r, b_r)
    h_o[...] = hn
    a = jnp.dot(hn, w1at_r[...], preferred_element_type=f32) + b1_r[...]
    a_o[...] = jnp.concatenate([a, jnp.zeros((NPAD - N, H), f32)], axis=0)
    b_o[...] = jnp.dot(hn, w1bt_r[...], preferred_element_type=f32)


def _tc_layer_final_body(h_r, accp_r, cnt_r, ew2t_r, eb2_r, nw1t_r, nb1_r,
                         nw2t_r, nb2_r, g_r, b_r, h_o):
    h_o[...] = _layer_core(h_r, accp_r, cnt_r, ew2t_r, eb2_r, nw1t_r,
                           nb1_r, nw2t_r, nb2_r, g_r, b_r)


def _tc_layer_ab(h, accp, cnt2d, ew2t, eb2, nw1t, nb1, nw2t, nb2, g, b,
                 w1at_n, b1_n, w1bt_n):
    return pl.pallas_call(
        _tc_layer_ab_body,
        out_shape=(jax.ShapeDtypeStruct((N, H), f32),
                   jax.ShapeDtypeStruct((NPAD, H), f32),
                   jax.ShapeDtypeStruct((N, H), f32)),
    )(h, accp, cnt2d, ew2t, eb2, nw1t, nb1, nw2t, nb2, g, b,
      w1at_n, b1_n, w1bt_n)


def _tc_layer_final(h, accp, cnt2d, ew2t, eb2, nw1t, nb1, nw2t, nb2, g, b):
    return pl.pallas_call(
        _tc_layer_final_body,
        out_shape=jax.ShapeDtypeStruct((N, H), f32),
    )(h, accp, cnt2d, ew2t, eb2, nw1t, nb1, nw2t, nb2, g, b)


def kernel(x, pos, edge_index, time_emb, t_w1, t_b1, t_w2, t_b2,
           edge_w1, edge_b1, edge_w2, edge_b2, node_w1, node_b1,
           node_w2, node_b2, ln_g, ln_b):
    row = edge_index[0]
    col = edge_index[1]
    px = pos[:, 0]
    py = pos[:, 1]
    pz = pos[:, 2]

    w1at = [edge_w1[l][:, :H].T for l in range(3)]
    w1bt = [edge_w1[l][:, H:2 * H].T for l in range(3)]
    w1c = [edge_w1[l][:, 2 * H] for l in range(3)]
    b1 = [edge_b1[l][None, :] for l in range(3)]
    ew2t = [edge_w2[l].T for l in range(3)]
    eb2 = [edge_b2[l][None, :] for l in range(3)]
    nw1t = [node_w1[l].T for l in range(3)]
    nb1 = [node_b1[l][None, :] for l in range(3)]
    nw2t = [node_w2[l].T for l in range(3)]
    nb2 = [node_b2[l][None, :] for l in range(3)]
    g = [ln_g[l][None, :] for l in range(3)]
    b = [ln_b[l][None, :] for l in range(3)]

    krow, kcol, kdist, kcnt, cnt = _sc_prep(row, col, px, py, pz)
    cnt2d = cnt.reshape(NPAD, 1)

    h, a_tab, b_tab = _tc_pre(x, time_emb, t_w1.T, t_b1[None, :], t_w2.T,
                              t_b2[None, :], w1at[0], b1[0], w1bt[0])

    for l in range(3):
        accf = _sc_edge(a_tab.reshape(NPAD * H), b_tab, krow, kcol, kdist,
                        kcnt, w1c[l])
        accp = accf.reshape(NPAD, H)
        if l < 2:
            h, a_tab, b_tab = _tc_layer_ab(
                h, accp, cnt2d, ew2t[l], eb2[l], nw1t[l], nb1[l], nw2t[l],
                nb2[l], g[l], b[l], w1at[l + 1], b1[l + 1], w1bt[l + 1])
        else:
            h = _tc_layer_final(
                h, accp, cnt2d, ew2t[l], eb2[l], nw1t[l], nb1[l], nw2t[l],
                nb2[l], g[l], b[l])
    return h, pos


# speedup vs baseline: 3.4267x; 3.4267x over previous
import jax
import jax.numpy as jnp
from jax import lax
from jax.experimental import pallas as pl
from jax.experimental.pallas import tpu as pltpu
from jax.experimental.pallas import tpu_sc as plsc

f32 = jnp.float32
i32 = jnp.int32

N = 10000
E = 320000
H = 128
NC = 2
NS = 16
NW = NC * NS
NR = 320
NPAD = NW * NR
CAP = 12320
SCH = 4000
NSCH = E // SCH
ECH = 80
CAPCH = CAP // ECH

_SC_PARAMS = pltpu.CompilerParams(needs_layout_passes=False)


def _silu(v):
    return v * jax.nn.sigmoid(v)


def _sc_mesh():
    return plsc.VectorSubcoreMesh(
        core_axis_name="c", subcore_axis_name="s",
        num_cores=NC, num_subcores=NS)


def _sc_prep_body(row_h, col_h, px_h, py_h, pz_h,
                  krow_h, kcol_h, kdist_h, kcnt_h, cnt_h,
                  pxv, pyv, pzv, rsc, csc, krow_st, kcol_st, kdist_st,
                  cntv, sbuf, sem_sc):
    w = lax.axis_index("c") * NS + lax.axis_index("s")
    lo = w * NR
    hi = lo + NR
    pltpu.sync_copy(px_h, pxv)
    pltpu.sync_copy(py_h, pyv)
    pltpu.sync_copy(pz_h, pzv)

    z16f = jnp.zeros((16,), f32)
    z16i = jnp.zeros((16,), i32)
    ones16 = jnp.ones((16,), f32)

    def zero_st(i, _):
        sl = pl.ds(i * 16, 16)
        krow_st[sl] = z16i
        kcol_st[sl] = z16i
        kdist_st[sl] = z16f
        return 0
    lax.fori_loop(0, CAP // 16, zero_st, 0)

    def zero_cnt(i, _):
        cntv[pl.ds(i * 16, 16)] = z16f
        return 0
    lax.fori_loop(0, NR // 16, zero_cnt, 0)

    def issue_scan(j, sl):
        off = j * SCH
        pltpu.async_copy(row_h.at[pl.ds(off, SCH)], rsc[sl], sem_sc[sl])
        pltpu.async_copy(col_h.at[pl.ds(off, SCH)], csc[sl], sem_sc[sl])

    def wait_scan(sl):
        pltpu.make_async_copy(row_h.at[pl.ds(0, SCH)], rsc[sl],
                              sem_sc[sl]).wait()
        pltpu.make_async_copy(col_h.at[pl.ds(0, SCH)], csc[sl],
                              sem_sc[sl]).wait()

    issue_scan(0, 0)

    def pair(p, cur):
        for u in range(2):
            j = p * 2 + u

            @pl.when(j + 1 < NSCH)
            def _():
                issue_scan(j + 1, 1 - u)
            wait_scan(u)

            def vec(v, cur):
                sl = pl.ds(v * 16, 16)
                r16 = rsc[u][sl]
                c16 = csc[u][sl]
                mask = jnp.logical_and(c16 >= lo, c16 < hi)
                plsc.addupdate_scatter(cntv, [c16 - lo], ones16, mask=mask)
                mp = plsc.all_reduce_population_count(mask)
                rank = plsc.cumsum(mask.astype(i32)) - 1
                addr = rank + cur
                plsc.store_scatter(krow_st, [addr], r16, mask=mask)
                plsc.store_scatter(kcol_st, [addr], c16, mask=mask)
                return jnp.minimum(cur + mp[0], CAP - 16)
            cur = lax.fori_loop(0, SCH // 16, vec, cur)
        return cur
    kept = lax.fori_loop(0, NSCH // 2, pair, jnp.int32(0))

    def dvec(v, _):
        sl = pl.ds(v * 16, 16)
        r16 = krow_st[sl]
        c16 = kcol_st[sl]
        dx = plsc.load_gather(pxv, [r16]) - plsc.load_gather(pxv, [c16])
        dy = plsc.load_gather(pyv, [r16]) - plsc.load_gather(pyv, [c16])
        dz = plsc.load_gather(pzv, [r16]) - plsc.load_gather(pzv, [c16])
        s = dx * dx + dy * dy + dz * dz + 1e-12
        y = plsc.bitcast(0x5F3759DF - (plsc.bitcast(s, i32) >> 1), f32)
        y = y * (1.5 - 0.5 * s * y * y)
        y = y * (1.5 - 0.5 * s * y * y)
        y = y * (1.5 - 0.5 * s * y * y)
        kdist_st[sl] = s * y
        return 0
    lax.fori_loop(0, CAP // 16, dvec, 0)

    pltpu.sync_copy(krow_st, krow_h.at[pl.ds(w * CAP, CAP)])
    pltpu.sync_copy(kcol_st, kcol_h.at[pl.ds(w * CAP, CAP)])
    pltpu.sync_copy(kdist_st, kdist_h.at[pl.ds(w * CAP, CAP)])
    pltpu.sync_copy(cntv, cnt_h.at[pl.ds(w * NR, NR)])
    sbuf[...] = jnp.full((16,), kept, dtype=i32)
    pltpu.sync_copy(sbuf, kcnt_h.at[pl.ds(w * 16, 16)])


def _sc_prep(row, col, px, py, pz):
    fn = pl.kernel(
        _sc_prep_body,
        out_type=(jax.ShapeDtypeStruct((NW * CAP,), i32),
                  jax.ShapeDtypeStruct((NW * CAP,), i32),
                  jax.ShapeDtypeStruct((NW * CAP,), f32),
                  jax.ShapeDtypeStruct((NW * 16,), i32),
                  jax.ShapeDtypeStruct((NPAD,), f32)),
        mesh=_sc_mesh(),
        compiler_params=_SC_PARAMS,
        scratch_types=[
            pltpu.VMEM((N,), f32), pltpu.VMEM((N,), f32), pltpu.VMEM((N,), f32),
            [pltpu.VMEM((SCH,), i32) for _ in range(2)],
            [pltpu.VMEM((SCH,), i32) for _ in range(2)],
            pltpu.VMEM((CAP,), i32), pltpu.VMEM((CAP,), i32),
            pltpu.VMEM((CAP,), f32),
            pltpu.VMEM((NR,), f32), pltpu.VMEM((16,), i32),
            [pltpu.SemaphoreType.DMA for _ in range(2)],
        ],
    )
    return fn(row, col, px, py, pz)


def _sc_edge_body(a_h, b_h, krow_h, kcol_h, kdist_h, kcnt_h, w1c_h, accp_h,
                  aloc, accl, bbuf, rowv, colv, distv, w1cv, cntb,
                  sem_i, sem_g):
    w = lax.axis_index("c") * NS + lax.axis_index("s")
    pltpu.sync_copy(w1c_h, w1cv)
    pltpu.sync_copy(a_h.at[pl.ds(w * NR * H, NR * H)], aloc)
    pltpu.sync_copy(kcnt_h.at[pl.ds(w * 16, 16)], cntb)
    kcnt = cntb[pl.ds(0, 16)][0]
    nch = (kcnt + (ECH - 1)) // ECH

    z16f = jnp.zeros((16,), f32)

    def zero_acc(i, _):
        accl[pl.ds(i * 16, 16)] = z16f
        return 0
    lax.fori_loop(0, (NR * H) // 16, zero_acc, 0)

    base = w * CAP

    def issue_idx(ch, sl):
        off = base + ch * ECH
        pltpu.async_copy(krow_h.at[pl.ds(off, ECH)], rowv[sl], sem_i[sl])
        pltpu.async_copy(kcol_h.at[pl.ds(off, ECH)], colv[sl], sem_i[sl])
        pltpu.async_copy(kdist_h.at[pl.ds(off, ECH)], distv[sl], sem_i[sl])

    def wait_idx(sl):
        pltpu.make_async_copy(krow_h.at[pl.ds(0, ECH)], rowv[sl],
                              sem_i[sl]).wait()
        pltpu.make_async_copy(kcol_h.at[pl.ds(0, ECH)], colv[sl],
                              sem_i[sl]).wait()
        pltpu.make_async_copy(kdist_h.at[pl.ds(0, ECH)], distv[sl],
                              sem_i[sl]).wait()

    def issue_gather(dsl, isl):
        pltpu.async_copy(b_h.at[rowv[isl]], bbuf[dsl], sem_g[dsl])

    def wait_gather(dsl, isl):
        pltpu.make_async_copy(b_h.at[rowv[isl]], bbuf[dsl],
                              sem_g[dsl]).wait()

    iota16 = lax.iota(i32, 16)
    lobase = w * NR

    def compute(c, dsl, isl):
        kb = jnp.minimum(kcnt - c * ECH, ECH)

        @plsc.parallel_loop(0, kb, unroll=2)
        def erow(e):
            e16 = jnp.full((16,), e, dtype=i32)
            lcv = plsc.load_gather(colv[isl], [e16]) - lobase
            d16 = plsc.load_gather(distv[isl], [e16])
            abase = lcv * H + iota16
            for k in range(H // 16):
                slc = pl.ds(16 * k, 16)
                addr = abase + (16 * k)
                av = plsc.load_gather(aloc, [addr])
                v = av + bbuf[dsl][e, slc] + d16 * w1cv[slc]
                d = 1.0 + jnp.exp(-jnp.maximum(v, -30.0))
                r = plsc.bitcast(0x7EF311C3 - plsc.bitcast(d, i32), f32)
                r = r * (2.0 - d * r)
                r = r * (2.0 - d * r)
                r = r * (2.0 - d * r)
                plsc.addupdate_scatter(accl, [addr], v * r)

    issue_idx(0, 0)
    issue_idx(1, 1)
    wait_idx(0)
    issue_gather(0, 0)

    def block(i, _):
        for u in range(6):
            c = i * 6 + u
            d0 = u % 2
            d1 = (u + 1) % 2
            i0 = u % 3
            i1 = (u + 1) % 3
            i2 = (u + 2) % 3

            @pl.when(c + 2 < nch)
            def _():
                issue_idx(c + 2, i2)

            @pl.when(c + 1 < nch)
            def _():
                wait_idx(i1)
                issue_gather(d1, i1)

            @pl.when(c < nch)
            def _():
                wait_gather(d0, i0)
                compute(c, d0, i0)
        return 0
    lax.fori_loop(0, (CAPCH + 6) // 6, block, 0)

    pltpu.sync_copy(accl, accp_h.at[pl.ds(w * NR * H, NR * H)])


def _sc_edge(a_flat, b_tab, krow, kcol, kdist, kcnt, w1c):
    fn = pl.kernel(
        _sc_edge_body,
        out_type=jax.ShapeDtypeStruct((NPAD * H,), f32),
        mesh=_sc_mesh(),
        compiler_params=_SC_PARAMS,
        scratch_types=[
            pltpu.VMEM((NR * H,), f32),
            pltpu.VMEM((NR * H,), f32),
            [pltpu.VMEM((ECH, H), f32) for _ in range(2)],
            [pltpu.VMEM((ECH,), i32) for _ in range(3)],
            [pltpu.VMEM((ECH,), i32) for _ in range(3)],
            [pltpu.VMEM((ECH,), f32) for _ in range(3)],
            pltpu.VMEM((H,), f32),
            pltpu.VMEM((16,), i32),
            [pltpu.SemaphoreType.DMA for _ in range(3)],
            [pltpu.SemaphoreType.DMA for _ in range(2)],
        ],
    )
    return fn(a_flat, b_tab, krow, kcol, kdist, kcnt, w1c)


def _tc_pre_body(x_r, temb_r, tw1t_r, tb1_r, tw2t_r, tb2_r,
                 w1at_r, b1_r, w1bt_r, h_o, a_o, b_o):
    t = _silu(jnp.dot(temb_r[...], tw1t_r[...], preferred_element_type=f32)
              + tb1_r[...])
    tp = jnp.dot(t, tw2t_r[...], preferred_element_type=f32) + tb2_r[...]
    h = x_r[...] + tp
    h_o[...] = h
    a = jnp.dot(h, w1at_r[...], preferred_element_type=f32) + b1_r[...]
    a_o[...] = jnp.concatenate([a, jnp.zeros((NPAD - N, H), f32)], axis=0)
    b_o[...] = jnp.dot(h, w1bt_r[...], preferred_element_type=f32)


def _tc_pre(x, time_emb, tw1t, tb1, tw2t, tb2, w1at, b1, w1bt):
    return pl.pallas_call(
        _tc_pre_body,
        out_shape=(jax.ShapeDtypeStruct((N, H), f32),
                   jax.ShapeDtypeStruct((NPAD, H), f32),
                   jax.ShapeDtypeStruct((N, H), f32)),
    )(x, time_emb, tw1t, tb1, tw2t, tb2, w1at, b1, w1bt)


def _layer_core(h_r, accp_r, cnt_r, ew2t_r, eb2_r, nw1t_r, nb1_r,
                nw2t_r, nb2_r, g_r, b_r):
    h = h_r[...]
    inv = 1.0 / jnp.maximum(cnt_r[...][:N], 1.0)
    acc = accp_r[...][:N] * inv
    agg = jnp.dot(acc, ew2t_r[...], preferred_element_type=f32) + eb2_r[...]
    t = _silu(jnp.dot(h, nw1t_r[...], preferred_element_type=f32) + nb1_r[...])
    nm = jnp.dot(t, nw2t_r[...], preferred_element_type=f32) + nb2_r[...]
    y = h + nm + agg
    mu = jnp.mean(y, axis=-1, keepdims=True)
    yc = y - mu
    var = jnp.mean(yc * yc, axis=-1, keepdims=True)
    return yc * lax.rsqrt(var + 1e-5) * g_r[...] + b_r[...]


def _tc_layer_ab_body(h_r, accp_r, cnt_r, ew2t_r, eb2_r, nw1t_r, nb1_r,
                      nw2t_r, nb2_r, g_r, b_r, w1at_r, b1_r, w1bt_r,
                      h_o, a_o, b_o):
    hn = _layer_core(h_r, accp_r, cnt_r, ew2t_r, eb2_r, nw1t_r, nb1_r,
                     nw2t_r, nb2_r, g_r, b_r)
    h_o[...] = hn
    a = jnp.dot(hn, w1at_r[...], preferred_element_type=f32) + b1_r[...]
    a_o[...] = jnp.concatenate([a, jnp.zeros((NPAD - N, H), f32)], axis=0)
    b_o[...] = jnp.dot(hn, w1bt_r[...], preferred_element_type=f32)


def _tc_layer_final_body(h_r, accp_r, cnt_r, ew2t_r, eb2_r, nw1t_r, nb1_r,
                         nw2t_r, nb2_r, g_r, b_r, h_o):
    h_o[...] = _layer_core(h_r, accp_r, cnt_r, ew2t_r, eb2_r, nw1t_r,
                           nb1_r, nw2t_r, nb2_r, g_r, b_r)


def _tc_layer_ab(h, accp, cnt2d, ew2t, eb2, nw1t, nb1, nw2t, nb2, g, b,
                 w1at_n, b1_n, w1bt_n):
    return pl.pallas_call(
        _tc_layer_ab_body,
        out_shape=(jax.ShapeDtypeStruct((N, H), f32),
                   jax.ShapeDtypeStruct((NPAD, H), f32),
                   jax.ShapeDtypeStruct((N, H), f32)),
    )(h, accp, cnt2d, ew2t, eb2, nw1t, nb1, nw2t, nb2, g, b,
      w1at_n, b1_n, w1bt_n)


def _tc_layer_final(h, accp, cnt2d, ew2t, eb2, nw1t, nb1, nw2t, nb2, g, b):
    return pl.pallas_call(
        _tc_layer_final_body,
        out_shape=jax.ShapeDtypeStruct((N, H), f32),
    )(h, accp, cnt2d, ew2t, eb2, nw1t, nb1, nw2t, nb2, g, b)


def kernel(x, pos, edge_index, time_emb, t_w1, t_b1, t_w2, t_b2,
           edge_w1, edge_b1, edge_w2, edge_b2, node_w1, node_b1,
           node_w2, node_b2, ln_g, ln_b):
    row = edge_index[0]
    col = edge_index[1]
    px = pos[:, 0]
    py = pos[:, 1]
    pz = pos[:, 2]

    w1at = [edge_w1[l][:, :H].T for l in range(3)]
    w1bt = [edge_w1[l][:, H:2 * H].T for l in range(3)]
    w1c = [edge_w1[l][:, 2 * H] for l in range(3)]
    b1 = [edge_b1[l][None, :] for l in range(3)]
    ew2t = [edge_w2[l].T for l in range(3)]
    eb2 = [edge_b2[l][None, :] for l in range(3)]
    nw1t = [node_w1[l].T for l in range(3)]
    nb1 = [node_b1[l][None, :] for l in range(3)]
    nw2t = [node_w2[l].T for l in range(3)]
    nb2 = [node_b2[l][None, :] for l in range(3)]
    g = [ln_g[l][None, :] for l in range(3)]
    b = [ln_b[l][None, :] for l in range(3)]

    krow, kcol, kdist, kcnt, cnt = _sc_prep(row, col, px, py, pz)
    cnt2d = cnt.reshape(NPAD, 1)

    h, a_tab, b_tab = _tc_pre(x, time_emb, t_w1.T, t_b1[None, :], t_w2.T,
                              t_b2[None, :], w1at[0], b1[0], w1bt[0])

    for l in range(3):
        accf = _sc_edge(a_tab.reshape(NPAD * H), b_tab, krow, kcol, kdist,
                        kcnt, w1c[l])
        accp = accf.reshape(NPAD, H)
        if l < 2:
            h, a_tab, b_tab = _tc_layer_ab(
                h, accp, cnt2d, ew2t[l], eb2[l], nw1t[l], nb1[l], nw2t[l],
                nb2[l], g[l], b[l], w1at[l + 1], b1[l + 1], w1bt[l + 1])
        else:
            h = _tc_layer_final(
                h, accp, cnt2d, ew2t[l], eb2[l], nw1t[l], nb1[l], nw2t[l],
                nb2[l], g[l], b[l])
    return h, pos


# speedup vs baseline: 3.7608x; 1.0975x over previous
import jax
import jax.numpy as jnp
from jax import lax
from jax.experimental import pallas as pl
from jax.experimental.pallas import tpu as pltpu
from jax.experimental.pallas import tpu_sc as plsc

f32 = jnp.float32
i32 = jnp.int32

N = 10000
E = 320000
H = 128
NC = 2
NS = 16
NW = NC * NS
NR = 320
NPAD = NW * NR
CAP = 12320
SCH = 4000
NSCH = E // SCH
ECH = 80
CAPCH = CAP // ECH

_SC_PARAMS = pltpu.CompilerParams(needs_layout_passes=False)


def _silu(v):
    return v * jax.nn.sigmoid(v)


def _sc_mesh():
    return plsc.VectorSubcoreMesh(
        core_axis_name="c", subcore_axis_name="s",
        num_cores=NC, num_subcores=NS)


def _sc_prep_body(row_h, col_h, px_h, py_h, pz_h,
                  krow_h, kcol_h, kdist_h, kcnt_h, cnt_h,
                  pxv, pyv, pzv, rsc, csc, krow_st, kcol_st, kdist_st,
                  cntv, sbuf, cnts, offs, sem_sc):
    w = lax.axis_index("c") * NS + lax.axis_index("s")
    lo = w * NR
    hi = lo + NR
    pltpu.sync_copy(px_h, pxv)
    pltpu.sync_copy(py_h, pyv)
    pltpu.sync_copy(pz_h, pzv)

    z16f = jnp.zeros((16,), f32)
    z16i = jnp.zeros((16,), i32)
    ones16 = jnp.ones((16,), f32)

    def zero_st(i, _):
        sl = pl.ds(i * 16, 16)
        krow_st[sl] = z16i
        kcol_st[sl] = z16i
        kdist_st[sl] = z16f
        return 0
    lax.fori_loop(0, CAP // 16, zero_st, 0)

    def zero_cnt(i, _):
        cntv[pl.ds(i * 16, 16)] = z16f
        return 0
    lax.fori_loop(0, NR // 16, zero_cnt, 0)

    def issue_scan(j, sl):
        off = j * SCH
        pltpu.async_copy(row_h.at[pl.ds(off, SCH)], rsc[sl], sem_sc[sl])
        pltpu.async_copy(col_h.at[pl.ds(off, SCH)], csc[sl], sem_sc[sl])

    def wait_scan(sl):
        pltpu.make_async_copy(row_h.at[pl.ds(0, SCH)], rsc[sl],
                              sem_sc[sl]).wait()
        pltpu.make_async_copy(col_h.at[pl.ds(0, SCH)], csc[sl],
                              sem_sc[sl]).wait()

    def zero_cnts(i, _):
        cnts[pl.ds(i * 16, 16)] = jnp.zeros((16,), i32)
        return 0
    lax.fori_loop(0, 16, zero_cnts, 0)

    issue_scan(0, 0)
    iota16 = lax.iota(i32, 16)
    lane0 = iota16 == 0
    NV = SCH // 16

    def pair(p, cur):
        for u in range(2):
            j = p * 2 + u

            @pl.when(j + 1 < NSCH)
            def _():
                issue_scan(j + 1, 1 - u)
            wait_scan(u)

            @plsc.parallel_loop(0, NV, unroll=2)
            def vec_a(v):
                sl = pl.ds(v * 16, 16)
                c16 = csc[u][sl]
                mask = jnp.logical_and(c16 >= lo, c16 < hi)
                plsc.addupdate_scatter(cntv, [c16 - lo], ones16, mask=mask)
                mp = plsc.all_reduce_population_count(mask)
                plsc.store_scatter(cnts, [jnp.full((16,), v, dtype=i32)],
                                   mp, mask=lane0)

            def prefix(q, carry):
                sl = pl.ds(q * 16, 16)
                c16 = cnts[sl]
                cs = plsc.cumsum(c16) + carry
                offs[sl] = cs - c16
                return cs[15]
            tot = lax.fori_loop(0, 16, prefix, jnp.int32(0))

            @plsc.parallel_loop(0, NV, unroll=2)
            def vec_b(v):
                sl = pl.ds(v * 16, 16)
                r16 = rsc[u][sl]
                c16 = csc[u][sl]
                mask = jnp.logical_and(c16 >= lo, c16 < hi)
                base = plsc.load_gather(offs, [jnp.full((16,), v, dtype=i32)])
                rank = plsc.cumsum(mask.astype(i32)) - 1
                addr = jnp.minimum(cur + base + rank, CAP - 1)
                plsc.store_scatter(krow_st, [addr], r16, mask=mask)
                plsc.store_scatter(kcol_st, [addr], c16, mask=mask)
            cur = jnp.minimum(cur + tot, CAP - 16)
        return cur
    kept = lax.fori_loop(0, NSCH // 2, pair, jnp.int32(0))

    @plsc.parallel_loop(0, CAP // 16, unroll=2)
    def dvec(v):
        sl = pl.ds(v * 16, 16)
        r16 = krow_st[sl]
        c16 = kcol_st[sl]
        dx = plsc.load_gather(pxv, [r16]) - plsc.load_gather(pxv, [c16])
        dy = plsc.load_gather(pyv, [r16]) - plsc.load_gather(pyv, [c16])
        dz = plsc.load_gather(pzv, [r16]) - plsc.load_gather(pzv, [c16])
        s = dx * dx + dy * dy + dz * dz + 1e-12
        y = plsc.bitcast(0x5F3759DF - (plsc.bitcast(s, i32) >> 1), f32)
        y = y * (1.5 - 0.5 * s * y * y)
        y = y * (1.5 - 0.5 * s * y * y)
        y = y * (1.5 - 0.5 * s * y * y)
        kdist_st[sl] = s * y

    pltpu.sync_copy(krow_st, krow_h.at[pl.ds(w * CAP, CAP)])
    pltpu.sync_copy(kcol_st, kcol_h.at[pl.ds(w * CAP, CAP)])
    pltpu.sync_copy(kdist_st, kdist_h.at[pl.ds(w * CAP, CAP)])
    pltpu.sync_copy(cntv, cnt_h.at[pl.ds(w * NR, NR)])
    sbuf[...] = jnp.full((16,), kept, dtype=i32)
    pltpu.sync_copy(sbuf, kcnt_h.at[pl.ds(w * 16, 16)])


def _sc_prep(row, col, px, py, pz):
    fn = pl.kernel(
        _sc_prep_body,
        out_type=(jax.ShapeDtypeStruct((NW * CAP,), i32),
                  jax.ShapeDtypeStruct((NW * CAP,), i32),
                  jax.ShapeDtypeStruct((NW * CAP,), f32),
                  jax.ShapeDtypeStruct((NW * 16,), i32),
                  jax.ShapeDtypeStruct((NPAD,), f32)),
        mesh=_sc_mesh(),
        compiler_params=_SC_PARAMS,
        scratch_types=[
            pltpu.VMEM((N,), f32), pltpu.VMEM((N,), f32), pltpu.VMEM((N,), f32),
            [pltpu.VMEM((SCH,), i32) for _ in range(2)],
            [pltpu.VMEM((SCH,), i32) for _ in range(2)],
            pltpu.VMEM((CAP,), i32), pltpu.VMEM((CAP,), i32),
            pltpu.VMEM((CAP,), f32),
            pltpu.VMEM((NR,), f32), pltpu.VMEM((16,), i32),
            pltpu.VMEM((256,), i32), pltpu.VMEM((256,), i32),
            [pltpu.SemaphoreType.DMA for _ in range(2)],
        ],
    )
    return fn(row, col, px, py, pz)


def _sc_edge_body(a_h, b_h, krow_h, kcol_h, kdist_h, kcnt_h, w1c_h, accp_h,
                  aloc, accl, bbuf, rowv, colv, distv, w1cv, cntb,
                  sem_i, sem_g):
    w = lax.axis_index("c") * NS + lax.axis_index("s")
    pltpu.sync_copy(w1c_h, w1cv)
    pltpu.sync_copy(a_h.at[pl.ds(w * NR * H, NR * H)], aloc)
    pltpu.sync_copy(kcnt_h.at[pl.ds(w * 16, 16)], cntb)
    kcnt = cntb[pl.ds(0, 16)][0]
    nch = (kcnt + (ECH - 1)) // ECH

    z16f = jnp.zeros((16,), f32)

    def zero_acc(i, _):
        accl[pl.ds(i * 16, 16)] = z16f
        return 0
    lax.fori_loop(0, (NR * H) // 16, zero_acc, 0)

    base = w * CAP

    def issue_idx(ch, sl):
        off = base + ch * ECH
        pltpu.async_copy(krow_h.at[pl.ds(off, ECH)], rowv[sl], sem_i[sl])
        pltpu.async_copy(kcol_h.at[pl.ds(off, ECH)], colv[sl], sem_i[sl])
        pltpu.async_copy(kdist_h.at[pl.ds(off, ECH)], distv[sl], sem_i[sl])

    def wait_idx(sl):
        pltpu.make_async_copy(krow_h.at[pl.ds(0, ECH)], rowv[sl],
                              sem_i[sl]).wait()
        pltpu.make_async_copy(kcol_h.at[pl.ds(0, ECH)], colv[sl],
                              sem_i[sl]).wait()
        pltpu.make_async_copy(kdist_h.at[pl.ds(0, ECH)], distv[sl],
                              sem_i[sl]).wait()

    def issue_gather(dsl, isl):
        pltpu.async_copy(b_h.at[rowv[isl]], bbuf[dsl], sem_g[dsl])

    def wait_gather(dsl, isl):
        pltpu.make_async_copy(b_h.at[rowv[isl]], bbuf[dsl],
                              sem_g[dsl]).wait()

    iota16 = lax.iota(i32, 16)
    lobase = w * NR

    def compute(c, dsl, isl):
        kb = jnp.minimum(kcnt - c * ECH, ECH)

        @plsc.parallel_loop(0, kb, unroll=2)
        def erow(e):
            e16 = jnp.full((16,), e, dtype=i32)
            lcv = plsc.load_gather(colv[isl], [e16]) - lobase
            d16 = plsc.load_gather(distv[isl], [e16])
            abase = lcv * H + iota16
            for k in range(H // 16):
                slc = pl.ds(16 * k, 16)
                addr = abase + (16 * k)
                av = plsc.load_gather(aloc, [addr])
                v = av + bbuf[dsl][e, slc] + d16 * w1cv[slc]
                d = 1.0 + jnp.exp(-jnp.maximum(v, -30.0))
                r = plsc.bitcast(0x7EF311C3 - plsc.bitcast(d, i32), f32)
                r = r * (2.0 - d * r)
                r = r * (2.0 - d * r)
                r = r * (2.0 - d * r)
                plsc.addupdate_scatter(accl, [addr], v * r)

    issue_idx(0, 0)
    issue_idx(1, 1)
    wait_idx(0)
    issue_gather(0, 0)

    def block(i, _):
        for u in range(6):
            c = i * 6 + u
            d0 = u % 2
            d1 = (u + 1) % 2
            i0 = u % 3
            i1 = (u + 1) % 3
            i2 = (u + 2) % 3

            @pl.when(c + 2 < nch)
            def _():
                issue_idx(c + 2, i2)

            @pl.when(c + 1 < nch)
            def _():
                wait_idx(i1)
                issue_gather(d1, i1)

            @pl.when(c < nch)
            def _():
                wait_gather(d0, i0)
                compute(c, d0, i0)
        return 0
    lax.fori_loop(0, (CAPCH + 6) // 6, block, 0)

    pltpu.sync_copy(accl, accp_h.at[pl.ds(w * NR * H, NR * H)])


def _sc_edge(a_flat, b_tab, krow, kcol, kdist, kcnt, w1c):
    fn = pl.kernel(
        _sc_edge_body,
        out_type=jax.ShapeDtypeStruct((NPAD * H,), f32),
        mesh=_sc_mesh(),
        compiler_params=_SC_PARAMS,
        scratch_types=[
            pltpu.VMEM((NR * H,), f32),
            pltpu.VMEM((NR * H,), f32),
            [pltpu.VMEM((ECH, H), f32) for _ in range(2)],
            [pltpu.VMEM((ECH,), i32) for _ in range(3)],
            [pltpu.VMEM((ECH,), i32) for _ in range(3)],
            [pltpu.VMEM((ECH,), f32) for _ in range(3)],
            pltpu.VMEM((H,), f32),
            pltpu.VMEM((16,), i32),
            [pltpu.SemaphoreType.DMA for _ in range(3)],
            [pltpu.SemaphoreType.DMA for _ in range(2)],
        ],
    )
    return fn(a_flat, b_tab, krow, kcol, kdist, kcnt, w1c)


def _tc_pre_body(x_r, temb_r, tw1t_r, tb1_r, tw2t_r, tb2_r,
                 w1at_r, b1_r, w1bt_r, h_o, a_o, b_o):
    t = _silu(jnp.dot(temb_r[...], tw1t_r[...], preferred_element_type=f32)
              + tb1_r[...])
    tp = jnp.dot(t, tw2t_r[...], preferred_element_type=f32) + tb2_r[...]
    h = x_r[...] + tp
    h_o[...] = h
    a = jnp.dot(h, w1at_r[...], preferred_element_type=f32) + b1_r[...]
    a_o[...] = jnp.concatenate([a, jnp.zeros((NPAD - N, H), f32)], axis=0)
    b_o[...] = jnp.dot(h, w1bt_r[...], preferred_element_type=f32)


def _tc_pre(x, time_emb, tw1t, tb1, tw2t, tb2, w1at, b1, w1bt):
    return pl.pallas_call(
        _tc_pre_body,
        out_shape=(jax.ShapeDtypeStruct((N, H), f32),
                   jax.ShapeDtypeStruct((NPAD, H), f32),
                   jax.ShapeDtypeStruct((N, H), f32)),
    )(x, time_emb, tw1t, tb1, tw2t, tb2, w1at, b1, w1bt)


def _layer_core(h_r, accp_r, cnt_r, ew2t_r, eb2_r, nw1t_r, nb1_r,
                nw2t_r, nb2_r, g_r, b_r):
    h = h_r[...]
    inv = 1.0 / jnp.maximum(cnt_r[...][:N], 1.0)
    acc = accp_r[...][:N] * inv
    agg = jnp.dot(acc, ew2t_r[...], preferred_element_type=f32) + eb2_r[...]
    t = _silu(jnp.dot(h, nw1t_r[...], preferred_element_type=f32) + nb1_r[...])
    nm = jnp.dot(t, nw2t_r[...], preferred_element_type=f32) + nb2_r[...]
    y = h + nm + agg
    mu = jnp.mean(y, axis=-1, keepdims=True)
    yc = y - mu
    var = jnp.mean(yc * yc, axis=-1, keepdims=True)
    return yc * lax.rsqrt(var + 1e-5) * g_r[...] + b_r[...]


def _tc_layer_ab_body(h_r, accp_r, cnt_r, ew2t_r, eb2_r, nw1t_r, nb1_r,
                      nw2t_r, nb2_r, g_r, b_r, w1at_r, b1_r, w1bt_r,
                      h_o, a_o, b_o):
    hn = _layer_core(h_r, accp_r, cnt_r, ew2t_r, eb2_r, nw1t_r, nb1_r,
                     nw2t_r, nb2_r, g_r, b_r)
    h_o[...] = hn
    a = jnp.dot(hn, w1at_r[...], preferred_element_type=f32) + b1_r[...]
    a_o[...] = jnp.concatenate([a, jnp.zeros((NPAD - N, H), f32)], axis=0)
    b_o[...] = jnp.dot(hn, w1bt_r[...], preferred_element_type=f32)


def _tc_layer_final_body(h_r, accp_r, cnt_r, ew2t_r, eb2_r, nw1t_r, nb1_r,
                         nw2t_r, nb2_r, g_r, b_r, h_o):
    h_o[...] = _layer_core(h_r, accp_r, cnt_r, ew2t_r, eb2_r, nw1t_r,
                           nb1_r, nw2t_r, nb2_r, g_r, b_r)


def _tc_layer_ab(h, accp, cnt2d, ew2t, eb2, nw1t, nb1, nw2t, nb2, g, b,
                 w1at_n, b1_n, w1bt_n):
    return pl.pallas_call(
        _tc_layer_ab_body,
        out_shape=(jax.ShapeDtypeStruct((N, H), f32),
                   jax.ShapeDtypeStruct((NPAD, H), f32),
                   jax.ShapeDtypeStruct((N, H), f32)),
    )(h, accp, cnt2d, ew2t, eb2, nw1t, nb1, nw2t, nb2, g, b,
      w1at_n, b1_n, w1bt_n)


def _tc_layer_final(h, accp, cnt2d, ew2t, eb2, nw1t, nb1, nw2t, nb2, g, b):
    return pl.pallas_call(
        _tc_layer_final_body,
        out_shape=jax.ShapeDtypeStruct((N, H), f32),
    )(h, accp, cnt2d, ew2t, eb2, nw1t, nb1, nw2t, nb2, g, b)


def kernel(x, pos, edge_index, time_emb, t_w1, t_b1, t_w2, t_b2,
           edge_w1, edge_b1, edge_w2, edge_b2, node_w1, node_b1,
           node_w2, node_b2, ln_g, ln_b):
    row = edge_index[0]
    col = edge_index[1]
    px = pos[:, 0]
    py = pos[:, 1]
    pz = pos[:, 2]

    w1at = [edge_w1[l][:, :H].T for l in range(3)]
    w1bt = [edge_w1[l][:, H:2 * H].T for l in range(3)]
    w1c = [edge_w1[l][:, 2 * H] for l in range(3)]
    b1 = [edge_b1[l][None, :] for l in range(3)]
    ew2t = [edge_w2[l].T for l in range(3)]
    eb2 = [edge_b2[l][None, :] for l in range(3)]
    nw1t = [node_w1[l].T for l in range(3)]
    nb1 = [node_b1[l][None, :] for l in range(3)]
    nw2t = [node_w2[l].T for l in range(3)]
    nb2 = [node_b2[l][None, :] for l in range(3)]
    g = [ln_g[l][None, :] for l in range(3)]
    b = [ln_b[l][None, :] for l in range(3)]

    krow, kcol, kdist, kcnt, cnt = _sc_prep(row, col, px, py, pz)
    cnt2d = cnt.reshape(NPAD, 1)

    h, a_tab, b_tab = _tc_pre(x, time_emb, t_w1.T, t_b1[None, :], t_w2.T,
                              t_b2[None, :], w1at[0], b1[0], w1bt[0])

    for l in range(3):
        accf = _sc_edge(a_tab.reshape(NPAD * H), b_tab, krow, kcol, kdist,
                        kcnt, w1c[l])
        accp = accf.reshape(NPAD, H)
        if l < 2:
            h, a_tab, b_tab = _tc_layer_ab(
                h, accp, cnt2d, ew2t[l], eb2[l], nw1t[l], nb1[l], nw2t[l],
                nb2[l], g[l], b[l], w1at[l + 1], b1[l + 1], w1bt[l + 1])
        else:
            h = _tc_layer_final(
                h, accp, cnt2d, ew2t[l], eb2[l], nw1t[l], nb1[l], nw2t[l],
                nb2[l], g[l], b[l])
    return h, pos


# speedup vs baseline: 4.0906x; 1.0877x over previous
import jax
import jax.numpy as jnp
from jax import lax
from jax.experimental import pallas as pl
from jax.experimental.pallas import tpu as pltpu
from jax.experimental.pallas import tpu_sc as plsc

f32 = jnp.float32
i32 = jnp.int32

N = 10000
E = 320000
H = 128
NC = 2
NS = 16
NW = NC * NS
NR = 320
NPAD = NW * NR
CAP = 12416
SCH = 4000
NSCH = E // SCH
ECH = 128
CAPCH = CAP // ECH

_SC_PARAMS = pltpu.CompilerParams(needs_layout_passes=False)


def _silu(v):
    return v * jax.nn.sigmoid(v)


def _sc_mesh():
    return plsc.VectorSubcoreMesh(
        core_axis_name="c", subcore_axis_name="s",
        num_cores=NC, num_subcores=NS)


def _sc_prep_body(row_h, col_h, px_h, py_h, pz_h,
                  krow_h, kcol_h, kdist_h, kcnt_h, cnt_h,
                  pxv, pyv, pzv, rsc, csc, krow_st, kcol_st, kdist_st,
                  cntv, sbuf, cnts, offs, sem_sc):
    w = lax.axis_index("c") * NS + lax.axis_index("s")
    lo = w * NR
    hi = lo + NR
    pltpu.sync_copy(px_h, pxv)
    pltpu.sync_copy(py_h, pyv)
    pltpu.sync_copy(pz_h, pzv)

    z16f = jnp.zeros((16,), f32)
    z16i = jnp.zeros((16,), i32)
    ones16 = jnp.ones((16,), f32)

    def zero_st(i, _):
        sl = pl.ds(i * 16, 16)
        krow_st[sl] = z16i
        kcol_st[sl] = z16i
        kdist_st[sl] = z16f
        return 0
    lax.fori_loop(0, CAP // 16, zero_st, 0)

    def zero_cnt(i, _):
        cntv[pl.ds(i * 16, 16)] = z16f
        return 0
    lax.fori_loop(0, NR // 16, zero_cnt, 0)

    def issue_scan(j, sl):
        off = j * SCH
        pltpu.async_copy(row_h.at[pl.ds(off, SCH)], rsc[sl], sem_sc[sl])
        pltpu.async_copy(col_h.at[pl.ds(off, SCH)], csc[sl], sem_sc[sl])

    def wait_scan(sl):
        pltpu.make_async_copy(row_h.at[pl.ds(0, SCH)], rsc[sl],
                              sem_sc[sl]).wait()
        pltpu.make_async_copy(col_h.at[pl.ds(0, SCH)], csc[sl],
                              sem_sc[sl]).wait()

    def zero_cnts(i, _):
        cnts[pl.ds(i * 16, 16)] = jnp.zeros((16,), i32)
        return 0
    lax.fori_loop(0, 16, zero_cnts, 0)

    issue_scan(0, 0)
    iota16 = lax.iota(i32, 16)
    lane0 = iota16 == 0
    NV = SCH // 16

    def pair(p, cur):
        for u in range(2):
            j = p * 2 + u

            @pl.when(j + 1 < NSCH)
            def _():
                issue_scan(j + 1, 1 - u)
            wait_scan(u)

            @plsc.parallel_loop(0, NV, unroll=2)
            def vec_a(v):
                sl = pl.ds(v * 16, 16)
                c16 = csc[u][sl]
                mask = jnp.logical_and(c16 >= lo, c16 < hi)
                plsc.addupdate_scatter(cntv, [c16 - lo], ones16, mask=mask)
                mp = plsc.all_reduce_population_count(mask)
                plsc.store_scatter(cnts, [jnp.full((16,), v, dtype=i32)],
                                   mp, mask=lane0)

            def prefix(q, carry):
                sl = pl.ds(q * 16, 16)
                c16 = cnts[sl]
                cs = plsc.cumsum(c16) + carry
                offs[sl] = cs - c16
                return cs[15]
            tot = lax.fori_loop(0, 16, prefix, jnp.int32(0))

            @plsc.parallel_loop(0, NV, unroll=2)
            def vec_b(v):
                sl = pl.ds(v * 16, 16)
                r16 = rsc[u][sl]
                c16 = csc[u][sl]
                mask = jnp.logical_and(c16 >= lo, c16 < hi)
                base = plsc.load_gather(offs, [jnp.full((16,), v, dtype=i32)])
                rank = plsc.cumsum(mask.astype(i32)) - 1
                addr = jnp.minimum(cur + base + rank, CAP - 1)
                plsc.store_scatter(krow_st, [addr], r16, mask=mask)
                plsc.store_scatter(kcol_st, [addr], c16, mask=mask)
            cur = jnp.minimum(cur + tot, CAP - 16)
        return cur
    kept = lax.fori_loop(0, NSCH // 2, pair, jnp.int32(0))

    @plsc.parallel_loop(0, CAP // 16, unroll=2)
    def dvec(v):
        sl = pl.ds(v * 16, 16)
        r16 = krow_st[sl]
        c16 = kcol_st[sl]
        dx = plsc.load_gather(pxv, [r16]) - plsc.load_gather(pxv, [c16])
        dy = plsc.load_gather(pyv, [r16]) - plsc.load_gather(pyv, [c16])
        dz = plsc.load_gather(pzv, [r16]) - plsc.load_gather(pzv, [c16])
        s = dx * dx + dy * dy + dz * dz + 1e-12
        y = plsc.bitcast(0x5F3759DF - (plsc.bitcast(s, i32) >> 1), f32)
        y = y * (1.5 - 0.5 * s * y * y)
        y = y * (1.5 - 0.5 * s * y * y)
        y = y * (1.5 - 0.5 * s * y * y)
        kdist_st[sl] = s * y

    pltpu.sync_copy(krow_st, krow_h.at[pl.ds(w * CAP, CAP)])
    pltpu.sync_copy(kcol_st, kcol_h.at[pl.ds(w * CAP, CAP)])
    pltpu.sync_copy(kdist_st, kdist_h.at[pl.ds(w * CAP, CAP)])
    pltpu.sync_copy(cntv, cnt_h.at[pl.ds(w * NR, NR)])
    sbuf[...] = jnp.full((16,), kept, dtype=i32)
    pltpu.sync_copy(sbuf, kcnt_h.at[pl.ds(w * 16, 16)])


def _sc_prep(row, col, px, py, pz):
    fn = pl.kernel(
        _sc_prep_body,
        out_type=(jax.ShapeDtypeStruct((NW * CAP,), i32),
                  jax.ShapeDtypeStruct((NW * CAP,), i32),
                  jax.ShapeDtypeStruct((NW * CAP,), f32),
                  jax.ShapeDtypeStruct((NW * 16,), i32),
                  jax.ShapeDtypeStruct((NPAD,), f32)),
        mesh=_sc_mesh(),
        compiler_params=_SC_PARAMS,
        scratch_types=[
            pltpu.VMEM((N,), f32), pltpu.VMEM((N,), f32), pltpu.VMEM((N,), f32),
            [pltpu.VMEM((SCH,), i32) for _ in range(2)],
            [pltpu.VMEM((SCH,), i32) for _ in range(2)],
            pltpu.VMEM((CAP,), i32), pltpu.VMEM((CAP,), i32),
            pltpu.VMEM((CAP,), f32),
            pltpu.VMEM((NR,), f32), pltpu.VMEM((16,), i32),
            pltpu.VMEM((256,), i32), pltpu.VMEM((256,), i32),
            [pltpu.SemaphoreType.DMA for _ in range(2)],
        ],
    )
    return fn(row, col, px, py, pz)


def _sc_edge_body(a_h, b_h, krow_h, kcol_h, kdist_h, kcnt_h, w1c_h, accp_h,
                  aloc, accl, bbuf, rowv, colv, distv, w1cv, cntb,
                  sem_i, sem_g):
    w = lax.axis_index("c") * NS + lax.axis_index("s")
    pltpu.sync_copy(w1c_h, w1cv)
    pltpu.sync_copy(a_h.at[pl.ds(w * NR * H, NR * H)], aloc)
    pltpu.sync_copy(kcnt_h.at[pl.ds(w * 16, 16)], cntb)
    kcnt = cntb[pl.ds(0, 16)][0]
    nch = (kcnt + (ECH - 1)) // ECH

    z16f = jnp.zeros((16,), f32)

    def zero_acc(i, _):
        accl[pl.ds(i * 16, 16)] = z16f
        return 0
    lax.fori_loop(0, (NR * H) // 16, zero_acc, 0)

    base = w * CAP

    def issue_idx(ch, sl):
        off = base + ch * ECH
        pltpu.async_copy(krow_h.at[pl.ds(off, ECH)], rowv[sl], sem_i[sl])
        pltpu.async_copy(kcol_h.at[pl.ds(off, ECH)], colv[sl], sem_i[sl])
        pltpu.async_copy(kdist_h.at[pl.ds(off, ECH)], distv[sl], sem_i[sl])

    def wait_idx(sl):
        pltpu.make_async_copy(krow_h.at[pl.ds(0, ECH)], rowv[sl],
                              sem_i[sl]).wait()
        pltpu.make_async_copy(kcol_h.at[pl.ds(0, ECH)], colv[sl],
                              sem_i[sl]).wait()
        pltpu.make_async_copy(kdist_h.at[pl.ds(0, ECH)], distv[sl],
                              sem_i[sl]).wait()

    def issue_gather(dsl, isl):
        pltpu.async_copy(b_h.at[rowv[isl]], bbuf[dsl], sem_g[dsl])

    def wait_gather(dsl, isl):
        pltpu.make_async_copy(b_h.at[rowv[isl]], bbuf[dsl],
                              sem_g[dsl]).wait()

    iota16 = lax.iota(i32, 16)
    lobase = w * NR

    def compute(c, dsl, isl):
        kb = jnp.minimum(kcnt - c * ECH, ECH)

        @plsc.parallel_loop(0, kb, unroll=2)
        def erow(e):
            e16 = jnp.full((16,), e, dtype=i32)
            lcv = plsc.load_gather(colv[isl], [e16]) - lobase
            d16 = plsc.load_gather(distv[isl], [e16])
            abase = lcv * H + iota16
            for k in range(H // 16):
                slc = pl.ds(16 * k, 16)
                addr = abase + (16 * k)
                av = plsc.load_gather(aloc, [addr])
                v = av + bbuf[dsl][e, slc] + d16 * w1cv[slc]
                d = 1.0 + jnp.exp(-jnp.maximum(v, -30.0))
                r = plsc.bitcast(0x7EF311C3 - plsc.bitcast(d, i32), f32)
                r = r * (2.0 - d * r)
                r = r * (2.0 - d * r)
                r = r * (2.0 - d * r)
                plsc.addupdate_scatter(accl, [addr], v * r)

    issue_idx(0, 0)
    issue_idx(1, 1)
    wait_idx(0)
    issue_gather(0, 0)

    def block(i, _):
        for u in range(6):
            c = i * 6 + u
            d0 = u % 2
            d1 = (u + 1) % 2
            i0 = u % 3
            i1 = (u + 1) % 3
            i2 = (u + 2) % 3

            @pl.when(c + 2 < nch)
            def _():
                issue_idx(c + 2, i2)

            @pl.when(c + 1 < nch)
            def _():
                wait_idx(i1)
                issue_gather(d1, i1)

            @pl.when(c < nch)
            def _():
                wait_gather(d0, i0)
                compute(c, d0, i0)
        return 0
    lax.fori_loop(0, (CAPCH + 6) // 6, block, 0)

    pltpu.sync_copy(accl, accp_h.at[pl.ds(w * NR * H, NR * H)])


def _sc_edge(a_flat, b_tab, krow, kcol, kdist, kcnt, w1c):
    fn = pl.kernel(
        _sc_edge_body,
        out_type=jax.ShapeDtypeStruct((NPAD * H,), f32),
        mesh=_sc_mesh(),
        compiler_params=_SC_PARAMS,
        scratch_types=[
            pltpu.VMEM((NR * H,), f32),
            pltpu.VMEM((NR * H,), f32),
            [pltpu.VMEM((ECH, H), f32) for _ in range(2)],
            [pltpu.VMEM((ECH,), i32) for _ in range(3)],
            [pltpu.VMEM((ECH,), i32) for _ in range(3)],
            [pltpu.VMEM((ECH,), f32) for _ in range(3)],
            pltpu.VMEM((H,), f32),
            pltpu.VMEM((16,), i32),
            [pltpu.SemaphoreType.DMA for _ in range(3)],
            [pltpu.SemaphoreType.DMA for _ in range(2)],
        ],
    )
    return fn(a_flat, b_tab, krow, kcol, kdist, kcnt, w1c)


def _tc_pre_body(x_r, temb_r, tw1t_r, tb1_r, tw2t_r, tb2_r,
                 w1at_r, b1_r, w1bt_r, h_o, a_o, b_o):
    t = _silu(jnp.dot(temb_r[...], tw1t_r[...], preferred_element_type=f32)
              + tb1_r[...])
    tp = jnp.dot(t, tw2t_r[...], preferred_element_type=f32) + tb2_r[...]
    h = x_r[...] + tp
    h_o[...] = h
    a = jnp.dot(h, w1at_r[...], preferred_element_type=f32) + b1_r[...]
    a_o[...] = jnp.concatenate([a, jnp.zeros((NPAD - N, H), f32)], axis=0)
    b_o[...] = jnp.dot(h, w1bt_r[...], preferred_element_type=f32)


def _tc_pre(x, time_emb, tw1t, tb1, tw2t, tb2, w1at, b1, w1bt):
    return pl.pallas_call(
        _tc_pre_body,
        out_shape=(jax.ShapeDtypeStruct((N, H), f32),
                   jax.ShapeDtypeStruct((NPAD, H), f32),
                   jax.ShapeDtypeStruct((N, H), f32)),
    )(x, time_emb, tw1t, tb1, tw2t, tb2, w1at, b1, w1bt)


def _layer_core(h_r, accp_r, cnt_r, ew2t_r, eb2_r, nw1t_r, nb1_r,
                nw2t_r, nb2_r, g_r, b_r):
    h = h_r[...]
    inv = 1.0 / jnp.maximum(cnt_r[...][:N], 1.0)
    acc = accp_r[...][:N] * inv
    agg = jnp.dot(acc, ew2t_r[...], preferred_element_type=f32) + eb2_r[...]
    t = _silu(jnp.dot(h, nw1t_r[...], preferred_element_type=f32) + nb1_r[...])
    nm = jnp.dot(t, nw2t_r[...], preferred_element_type=f32) + nb2_r[...]
    y = h + nm + agg
    mu = jnp.mean(y, axis=-1, keepdims=True)
    yc = y - mu
    var = jnp.mean(yc * yc, axis=-1, keepdims=True)
    return yc * lax.rsqrt(var + 1e-5) * g_r[...] + b_r[...]


def _tc_layer_ab_body(h_r, accp_r, cnt_r, ew2t_r, eb2_r, nw1t_r, nb1_r,
                      nw2t_r, nb2_r, g_r, b_r, w1at_r, b1_r, w1bt_r,
                      h_o, a_o, b_o):
    hn = _layer_core(h_r, accp_r, cnt_r, ew2t_r, eb2_r, nw1t_r, nb1_r,
                     nw2t_r, nb2_r, g_r, b_r)
    h_o[...] = hn
    a = jnp.dot(hn, w1at_r[...], preferred_element_type=f32) + b1_r[...]
    a_o[...] = jnp.concatenate([a, jnp.zeros((NPAD - N, H), f32)], axis=0)
    b_o[...] = jnp.dot(hn, w1bt_r[...], preferred_element_type=f32)


def _tc_layer_final_body(h_r, accp_r, cnt_r, ew2t_r, eb2_r, nw1t_r, nb1_r,
                         nw2t_r, nb2_r, g_r, b_r, h_o):
    h_o[...] = _layer_core(h_r, accp_r, cnt_r, ew2t_r, eb2_r, nw1t_r,
                           nb1_r, nw2t_r, nb2_r, g_r, b_r)


def _tc_layer_ab(h, accp, cnt2d, ew2t, eb2, nw1t, nb1, nw2t, nb2, g, b,
                 w1at_n, b1_n, w1bt_n):
    return pl.pallas_call(
        _tc_layer_ab_body,
        out_shape=(jax.ShapeDtypeStruct((N, H), f32),
                   jax.ShapeDtypeStruct((NPAD, H), f32),
                   jax.ShapeDtypeStruct((N, H), f32)),
    )(h, accp, cnt2d, ew2t, eb2, nw1t, nb1, nw2t, nb2, g, b,
      w1at_n, b1_n, w1bt_n)


def _tc_layer_final(h, accp, cnt2d, ew2t, eb2, nw1t, nb1, nw2t, nb2, g, b):
    return pl.pallas_call(
        _tc_layer_final_body,
        out_shape=jax.ShapeDtypeStruct((N, H), f32),
    )(h, accp, cnt2d, ew2t, eb2, nw1t, nb1, nw2t, nb2, g, b)


def kernel(x, pos, edge_index, time_emb, t_w1, t_b1, t_w2, t_b2,
           edge_w1, edge_b1, edge_w2, edge_b2, node_w1, node_b1,
           node_w2, node_b2, ln_g, ln_b):
    row = edge_index[0]
    col = edge_index[1]
    px = pos[:, 0]
    py = pos[:, 1]
    pz = pos[:, 2]

    w1at = [edge_w1[l][:, :H].T for l in range(3)]
    w1bt = [edge_w1[l][:, H:2 * H].T for l in range(3)]
    w1c = [edge_w1[l][:, 2 * H] for l in range(3)]
    b1 = [edge_b1[l][None, :] for l in range(3)]
    ew2t = [edge_w2[l].T for l in range(3)]
    eb2 = [edge_b2[l][None, :] for l in range(3)]
    nw1t = [node_w1[l].T for l in range(3)]
    nb1 = [node_b1[l][None, :] for l in range(3)]
    nw2t = [node_w2[l].T for l in range(3)]
    nb2 = [node_b2[l][None, :] for l in range(3)]
    g = [ln_g[l][None, :] for l in range(3)]
    b = [ln_b[l][None, :] for l in range(3)]

    krow, kcol, kdist, kcnt, cnt = _sc_prep(row, col, px, py, pz)
    cnt2d = cnt.reshape(NPAD, 1)

    h, a_tab, b_tab = _tc_pre(x, time_emb, t_w1.T, t_b1[None, :], t_w2.T,
                              t_b2[None, :], w1at[0], b1[0], w1bt[0])

    for l in range(3):
        accf = _sc_edge(a_tab.reshape(NPAD * H), b_tab, krow, kcol, kdist,
                        kcnt, w1c[l])
        accp = accf.reshape(NPAD, H)
        if l < 2:
            h, a_tab, b_tab = _tc_layer_ab(
                h, accp, cnt2d, ew2t[l], eb2[l], nw1t[l], nb1[l], nw2t[l],
                nb2[l], g[l], b[l], w1at[l + 1], b1[l + 1], w1bt[l + 1])
        else:
            h = _tc_layer_final(
                h, accp, cnt2d, ew2t[l], eb2[l], nw1t[l], nb1[l], nw2t[l],
                nb2[l], g[l], b[l])
    return h, pos
